# Initial kernel scaffold; baseline (speedup 1.0000x reference)
import jax
import jax.numpy as jnp
from jax.experimental import pallas as pl


def kernel(x, emb_table, lin_table, lin_bias, W1, b1, W2, b2, W3, b3):
    raise NotImplementedError("write your pallas kernel here")


# speedup vs baseline: 20.8171x; 20.8171x over previous
import jax
import jax.numpy as jnp
from jax import lax
from jax.experimental import pallas as pl
from jax.experimental.pallas import tpu as pltpu
from jax.experimental.pallas import tpu_sc as plsc

_B = 16384
_F = 26
_FIELD_DIM = 1000
_D = 128
_H1 = 1024
_H2 = 512

_NC = 2
_NS = 16
_NW = _NC * _NS
_RPW = _B // _NW
_GB = 4
_GI = _GB * _F
_NG = _RPW // _GB
_SB = 16
_NSUP = _RPW // _SB


def _sc_body(idx_hbm, emb_hbm, lin_hbm, cross_hbm, linflat_hbm,
             idx_v, rows_v, linv_v, cross_v, sem_e, sem_l):
    wid = lax.axis_index("s") * _NC + lax.axis_index("c")
    pltpu.sync_copy(idx_hbm.at[pl.ds(wid * _NG, _NG)], idx_v)

    @pl.loop(0, _NSUP)
    def _super(k):
        for jj in range(_SB // _GB):
            j = k * (_SB // _GB) + jj
            ce = pltpu.async_copy(emb_hbm.at[idx_v.at[j]], rows_v, sem_e)
            cl = pltpu.async_copy(lin_hbm.at[idx_v.at[j]],
                                  linv_v.at[pl.ds(jj * _GI, _GI)], sem_l)
            ce.wait()
            cl.wait()

            @pl.loop(0, _GB)
            def _row(r):
                base = r * _F
                acc_s = [jnp.zeros((16,), jnp.float32) for _ in range(8)]
                acc_q = [jnp.zeros((16,), jnp.float32) for _ in range(8)]
                for f in range(_F):
                    for d in range(8):
                        v = rows_v[base + f, pl.ds(d * 16, 16)]
                        acc_s[d] = acc_s[d] + v
                        acc_q[d] = acc_q[d] + v * v
                for d in range(8):
                    cross_v[jj * _GB + r, pl.ds(d * 16, 16)] = (
                        0.5 * (acc_s[d] * acc_s[d] - acc_q[d]))

        ob = wid * _RPW + k * _SB
        pltpu.sync_copy(cross_v, cross_hbm.at[pl.ds(ob, _SB)])
        pltpu.sync_copy(linv_v, linflat_hbm.at[pl.ds(ob * _F, _SB * _F)])


def _sc_gather(idx, emb, lin):
    mesh = plsc.VectorSubcoreMesh(core_axis_name="c", subcore_axis_name="s")
    f = pl.kernel(
        _sc_body,
        out_type=[jax.ShapeDtypeStruct((_B, _D), jnp.float32),
                  jax.ShapeDtypeStruct((_B * _F,), jnp.float32)],
        mesh=mesh,
        scratch_types=[
            pltpu.VMEM((_NG, _GI), jnp.int32),
            pltpu.VMEM((_GI, _D), jnp.float32),
            pltpu.VMEM(((_SB // _GB) * _GI,), jnp.float32),
            pltpu.VMEM((_SB, _D), jnp.float32),
            pltpu.SemaphoreType.DMA,
            pltpu.SemaphoreType.DMA,
        ],
    )
    return f(idx, emb, lin)


def _mlp(cross, lin_vals, bias0, w1, b1, w2, b2, w3row):
    bm = 512

    def body(b0_ref, c_ref, l_ref, w1_ref, b1_ref, w2_ref, b2_ref, w3_ref,
             o_ref):
        x = c_ref[...].astype(jnp.bfloat16)
        h = jnp.dot(x, w1_ref[...], preferred_element_type=jnp.float32)
        h = jnp.maximum(h + b1_ref[...][None, :], 0.0).astype(jnp.bfloat16)
        h = jnp.dot(h, w2_ref[...], preferred_element_type=jnp.float32)
        h = jnp.maximum(h + b2_ref[...][None, :], 0.0)
        o = jnp.sum(h * w3_ref[...], axis=1)
        o = o + jnp.sum(l_ref[...], axis=1) + b0_ref[0]
        o_ref[...] = jax.nn.sigmoid(o)

    return pl.pallas_call(
        body,
        grid=(_B // bm,),
        in_specs=[
            pl.BlockSpec(memory_space=pltpu.SMEM),
            pl.BlockSpec((bm, _D), lambda i: (i, 0)),
            pl.BlockSpec((bm, _F), lambda i: (i, 0)),
            pl.BlockSpec((_D, _H1), lambda i: (0, 0)),
            pl.BlockSpec((_H1,), lambda i: (0,)),
            pl.BlockSpec((_H1, _H2), lambda i: (0, 0)),
            pl.BlockSpec((_H2,), lambda i: (0,)),
            pl.BlockSpec((1, _H2), lambda i: (0, 0)),
        ],
        out_specs=pl.BlockSpec((bm,), lambda i: (i,)),
        out_shape=jax.ShapeDtypeStruct((_B,), jnp.float32),
    )(bias0, cross, lin_vals, w1, b1, w2, b2, w3row)


def kernel(x, emb_table, lin_table, lin_bias, W1, b1, W2, b2, W3, b3):
    offs = (jnp.arange(_F, dtype=x.dtype) * _FIELD_DIM)[None, :]
    idx = (x + offs).astype(jnp.int32).reshape(_B // _GB, _GI)
    cross, linflat = _sc_gather(idx, emb_table, lin_table[:, 0])
    bias0 = (lin_bias + b3).astype(jnp.float32)
    return _mlp(cross, linflat.reshape(_B, _F), bias0,
                W1.astype(jnp.bfloat16), b1, W2.astype(jnp.bfloat16), b2,
                W3.reshape(1, _H2).astype(jnp.float32))


# speedup vs baseline: 26.4668x; 1.2714x over previous
import jax
import jax.numpy as jnp
from jax import lax
from jax.experimental import pallas as pl
from jax.experimental.pallas import tpu as pltpu
from jax.experimental.pallas import tpu_sc as plsc

_B = 16384
_F = 26
_FIELD_DIM = 1000
_D = 128
_H1 = 1024
_H2 = 512

_NC = 2
_NS = 16
_NW = _NC * _NS
_RPW = _B // _NW
_GB = 4
_GI = _GB * _F
_NG = _RPW // _GB
_SB = 16
_NSUP = _RPW // _SB


def _sc_body(idx_hbm, emb_hbm, lin_hbm, cross_hbm, linflat_hbm,
             idx_v, rows0_v, rows1_v, linv_v, cross_v,
             sem_e0, sem_e1, sem_l0, sem_l1):
    wid = lax.axis_index("s") * _NC + lax.axis_index("c")
    pltpu.sync_copy(idx_hbm.at[pl.ds(wid * _NG, _NG)], idx_v)
    rows = (rows0_v, rows1_v)
    sems_e = (sem_e0, sem_e1)
    sems_l = (sem_l0, sem_l1)

    def start(j, p):
        pltpu.async_copy(emb_hbm.at[idx_v.at[j]], rows[p], sems_e[p])
        slot = lax.rem(j, 8)
        pltpu.async_copy(lin_hbm.at[idx_v.at[j]],
                         linv_v.at[pl.ds(slot * _GI, _GI)], sems_l[p])

    start(0, 0)

    @pl.loop(0, _NSUP)
    def _super(k):
        for jj in range(_SB // _GB):
            j = k * (_SB // _GB) + jj
            p = jj % 2
            pltpu.make_async_copy(emb_hbm.at[idx_v.at[j]], rows[p],
                                  sems_e[p]).wait()
            pltpu.make_async_copy(lin_hbm.at[idx_v.at[j]],
                                  linv_v.at[pl.ds(0, _GI)],
                                  sems_l[p]).wait()

            @pl.when(j + 1 < _NG)
            def _():
                start(j + 1, 1 - p)

            rv = rows[p]

            @pl.loop(0, _GB)
            def _row(r):
                base = r * _F
                acc_s = [jnp.zeros((16,), jnp.float32) for _ in range(8)]
                acc_q = [jnp.zeros((16,), jnp.float32) for _ in range(8)]
                for f in range(_F):
                    for d in range(8):
                        v = rv[base + f, pl.ds(d * 16, 16)]
                        acc_s[d] = acc_s[d] + v
                        acc_q[d] = acc_q[d] + v * v
                for d in range(8):
                    cross_v[jj * _GB + r, pl.ds(d * 16, 16)] = (
                        0.5 * (acc_s[d] * acc_s[d] - acc_q[d]))

        ob = wid * _RPW + k * _SB
        pltpu.sync_copy(cross_v, cross_hbm.at[pl.ds(ob, _SB)])
        half = lax.rem(k, 2) * (_SB * _F)
        pltpu.sync_copy(linv_v.at[pl.ds(half, _SB * _F)],
                        linflat_hbm.at[pl.ds(ob * _F, _SB * _F)])


def _sc_gather(idx, emb, lin):
    mesh = plsc.VectorSubcoreMesh(core_axis_name="c", subcore_axis_name="s")
    f = pl.kernel(
        _sc_body,
        out_type=[jax.ShapeDtypeStruct((_B, _D), jnp.float32),
                  jax.ShapeDtypeStruct((_B * _F,), jnp.float32)],
        mesh=mesh,
        scratch_types=[
            pltpu.VMEM((_NG, _GI), jnp.int32),
            pltpu.VMEM((_GI, _D), jnp.float32),
            pltpu.VMEM((_GI, _D), jnp.float32),
            pltpu.VMEM((2 * _SB * _F,), jnp.float32),
            pltpu.VMEM((_SB, _D), jnp.float32),
            pltpu.SemaphoreType.DMA,
            pltpu.SemaphoreType.DMA,
            pltpu.SemaphoreType.DMA,
            pltpu.SemaphoreType.DMA,
        ],
        compiler_params=pltpu.CompilerParams(needs_layout_passes=False),
    )
    return f(idx, emb, lin)


def _mlp(cross, lin_vals, bias0, w1, b1, w2, b2, w3row):
    bm = 512

    def body(b0_ref, c_ref, l_ref, w1_ref, b1_ref, w2_ref, b2_ref, w3_ref,
             o_ref):
        x = c_ref[...].astype(jnp.bfloat16)
        h = jnp.dot(x, w1_ref[...], preferred_element_type=jnp.float32)
        h = jnp.maximum(h + b1_ref[...][None, :], 0.0).astype(jnp.bfloat16)
        h = jnp.dot(h, w2_ref[...], preferred_element_type=jnp.float32)
        h = jnp.maximum(h + b2_ref[...][None, :], 0.0)
        o = jnp.sum(h * w3_ref[...], axis=1)
        o = o + jnp.sum(l_ref[...], axis=1) + b0_ref[0]
        o_ref[...] = jax.nn.sigmoid(o)

    return pl.pallas_call(
        body,
        grid=(_B // bm,),
        in_specs=[
            pl.BlockSpec(memory_space=pltpu.SMEM),
            pl.BlockSpec((bm, _D), lambda i: (i, 0)),
            pl.BlockSpec((bm, _F), lambda i: (i, 0)),
            pl.BlockSpec((_D, _H1), lambda i: (0, 0)),
            pl.BlockSpec((_H1,), lambda i: (0,)),
            pl.BlockSpec((_H1, _H2), lambda i: (0, 0)),
            pl.BlockSpec((_H2,), lambda i: (0,)),
            pl.BlockSpec((1, _H2), lambda i: (0, 0)),
        ],
        out_specs=pl.BlockSpec((bm,), lambda i: (i,)),
        out_shape=jax.ShapeDtypeStruct((_B,), jnp.float32),
    )(bias0, cross, lin_vals, w1, b1, w2, b2, w3row)


def kernel(x, emb_table, lin_table, lin_bias, W1, b1, W2, b2, W3, b3):
    offs = (jnp.arange(_F, dtype=x.dtype) * _FIELD_DIM)[None, :]
    idx = (x + offs).astype(jnp.int32).reshape(_B // _GB, _GI)
    cross, linflat = _sc_gather(idx, emb_table, lin_table[:, 0])
    bias0 = (lin_bias + b3).astype(jnp.float32)
    return _mlp(cross, linflat.reshape(_B, _F), bias0,
                W1.astype(jnp.bfloat16), b1, W2.astype(jnp.bfloat16), b2,
                W3.reshape(1, _H2).astype(jnp.float32))


# speedup vs baseline: 36.1104x; 1.3644x over previous
import jax
import jax.numpy as jnp
from jax import lax
from jax.experimental import pallas as pl
from jax.experimental.pallas import tpu as pltpu
from jax.experimental.pallas import tpu_sc as plsc

_B = 16384
_F = 26
_FIELD_DIM = 1000
_D = 128
_H1 = 1024
_H2 = 512

_NC = 2
_NS = 16
_NW = _NC * _NS
_RPW = _B // _NW
_GB = 4
_GI = _GB * _F
_NG = _RPW // _GB
_SB = 16
_NSUP = _RPW // _SB


def _sc_body(idx_hbm, emb_hbm, lin_hbm, cross_hbm, linflat_hbm,
             idx_v, rows0_v, rows1_v, rows2_v, rows3_v, linall_v, cross_v,
             sem_e0, sem_e1, sem_e2, sem_e3, sem_l):
    wid = lax.axis_index("s") * _NC + lax.axis_index("c")
    pltpu.sync_copy(idx_hbm.at[pl.ds(wid * _NG, _NG)], idx_v)
    rows = (rows0_v, rows1_v, rows2_v, rows3_v)
    sems_e = (sem_e0, sem_e1, sem_e2, sem_e3)

    def start(j, p):
        pltpu.async_copy(emb_hbm.at[idx_v.at[j]], rows[p], sems_e[p])
        pltpu.async_copy(lin_hbm.at[idx_v.at[j]],
                         linall_v.at[pl.ds(j * _GI, _GI)], sem_l)

    for p in range(4):
        start(p, p)

    @pl.loop(0, _NSUP)
    def _super(k):
        for jj in range(_SB // _GB):
            j = k * (_SB // _GB) + jj
            pltpu.make_async_copy(emb_hbm.at[idx_v.at[j]], rows[jj],
                                  sems_e[jj]).wait()

            @pl.when(j + 4 < _NG)
            def _():
                start(j + 4, jj)

            rv = rows[jj]

            @pl.loop(0, _GB)
            def _row(r):
                base = r * _F
                acc_s = [jnp.zeros((16,), jnp.float32) for _ in range(8)]
                acc_q = [jnp.zeros((16,), jnp.float32) for _ in range(8)]
                for f in range(_F):
                    for d in range(8):
                        v = rv[base + f, pl.ds(d * 16, 16)]
                        acc_s[d] = acc_s[d] + v
                        acc_q[d] = acc_q[d] + v * v
                for d in range(8):
                    cross_v[jj * _GB + r, pl.ds(d * 16, 16)] = (
                        0.5 * (acc_s[d] * acc_s[d] - acc_q[d]))

        ob = wid * _RPW + k * _SB
        pltpu.sync_copy(cross_v, cross_hbm.at[pl.ds(ob, _SB)])

    pltpu.make_async_copy(lin_hbm.at[pl.ds(0, _RPW * _F)], linall_v,
                          sem_l).wait()
    pltpu.sync_copy(linall_v, linflat_hbm.at[pl.ds(wid * _RPW * _F,
                                                   _RPW * _F)])


def _sc_gather(idx, emb, lin):
    mesh = plsc.VectorSubcoreMesh(core_axis_name="c", subcore_axis_name="s")
    f = pl.kernel(
        _sc_body,
        out_type=[jax.ShapeDtypeStruct((_B, _D), jnp.float32),
                  jax.ShapeDtypeStruct((_B * _F,), jnp.float32)],
        mesh=mesh,
        scratch_types=[
            pltpu.VMEM((_NG, _GI), jnp.int32),
            pltpu.VMEM((_GI, _D), jnp.float32),
            pltpu.VMEM((_GI, _D), jnp.float32),
            pltpu.VMEM((_GI, _D), jnp.float32),
            pltpu.VMEM((_GI, _D), jnp.float32),
            pltpu.VMEM((_RPW * _F,), jnp.float32),
            pltpu.VMEM((_SB, _D), jnp.float32),
            pltpu.SemaphoreType.DMA,
            pltpu.SemaphoreType.DMA,
            pltpu.SemaphoreType.DMA,
            pltpu.SemaphoreType.DMA,
            pltpu.SemaphoreType.DMA,
        ],
        compiler_params=pltpu.CompilerParams(needs_layout_passes=False),
    )
    return f(idx, emb, lin)


def _mlp(cross, lin_vals, bias0, w1, b1, w2, b2, w3row):
    bm = 512

    def body(b0_ref, c_ref, l_ref, w1_ref, b1_ref, w2_ref, b2_ref, w3_ref,
             o_ref):
        x = c_ref[...].astype(jnp.bfloat16)
        h = jnp.dot(x, w1_ref[...], preferred_element_type=jnp.float32)
        h = jnp.maximum(h + b1_ref[...][None, :], 0.0).astype(jnp.bfloat16)
        h = jnp.dot(h, w2_ref[...], preferred_element_type=jnp.float32)
        h = jnp.maximum(h + b2_ref[...][None, :], 0.0)
        o = jnp.sum(h * w3_ref[...], axis=1)
        o = o + jnp.sum(l_ref[...], axis=1) + b0_ref[0]
        o_ref[...] = jax.nn.sigmoid(o)

    return pl.pallas_call(
        body,
        grid=(_B // bm,),
        in_specs=[
            pl.BlockSpec(memory_space=pltpu.SMEM),
            pl.BlockSpec((bm, _D), lambda i: (i, 0)),
            pl.BlockSpec((bm, _F), lambda i: (i, 0)),
            pl.BlockSpec((_D, _H1), lambda i: (0, 0)),
            pl.BlockSpec((_H1,), lambda i: (0,)),
            pl.BlockSpec((_H1, _H2), lambda i: (0, 0)),
            pl.BlockSpec((_H2,), lambda i: (0,)),
            pl.BlockSpec((1, _H2), lambda i: (0, 0)),
        ],
        out_specs=pl.BlockSpec((bm,), lambda i: (i,)),
        out_shape=jax.ShapeDtypeStruct((_B,), jnp.float32),
    )(bias0, cross, lin_vals, w1, b1, w2, b2, w3row)


def kernel(x, emb_table, lin_table, lin_bias, W1, b1, W2, b2, W3, b3):
    offs = (jnp.arange(_F, dtype=x.dtype) * _FIELD_DIM)[None, :]
    idx = (x + offs).astype(jnp.int32).reshape(_B // _GB, _GI)
    cross, linflat = _sc_gather(idx, emb_table, lin_table[:, 0])
    bias0 = (lin_bias + b3).astype(jnp.float32)
    return _mlp(cross, linflat.reshape(_B, _F), bias0,
                W1.astype(jnp.bfloat16), b1, W2.astype(jnp.bfloat16), b2,
                W3.reshape(1, _H2).astype(jnp.float32))
